# Initial kernel scaffold; baseline (speedup 1.0000x reference)
import jax
import jax.numpy as jnp
from jax.experimental import pallas as pl


def kernel(x, z, z_table, W, b):
    raise NotImplementedError("write your pallas kernel here")


# speedup vs baseline: 2.1698x; 2.1698x over previous
import dataclasses
import functools

import jax
import jax.numpy as jnp
from jax import lax
from jax.experimental import pallas as pl
from jax.experimental.pallas import tpu as pltpu
from jax.experimental.pallas import tpu_sc as plsc

N = 1_000_000
HIDDEN = 256
EMB = 8
XDIM = 11
VOCAB = 1000
TROWS = (VOCAB * EMB + 127) // 128
TROWS_PAD = 64

SC_CORES = 2
SC_SUBCORES = 16
SC_LANES = 16
SC_BLOCK = 1024
SC_CHUNK = SC_BLOCK * SC_CORES * SC_SUBCORES
N_PAD = ((N + SC_CHUNK - 1) // SC_CHUNK) * SC_CHUNK

BT = 2000


def _sc_gather(table_packed, idx2d):
    mesh = plsc.VectorSubcoreMesh(core_axis_name="core",
                                  subcore_axis_name="subcore")
    cp = pltpu.CompilerParams()
    if "needs_layout_passes" in pltpu.CompilerParams.__dataclass_fields__:
        cp = dataclasses.replace(cp, needs_layout_passes=False)

    @functools.partial(
        pl.kernel,
        out_type=jax.ShapeDtypeStruct((N_PAD // 16, 128), jnp.float32),
        mesh=mesh,
        scratch_types=[pltpu.VMEM((TROWS_PAD, 128), jnp.float32)],
        compiler_params=cp,
    )
    def gather_kernel(table_hbm, idx_hbm, out_hbm, table_v):
        pltpu.sync_copy(table_hbm, table_v)

        lanes = lax.iota(jnp.int32, SC_LANES)
        row_off = lanes >> 3
        cold = lanes & 7
        zero = jnp.zeros((SC_LANES,), jnp.int32)

        def body(idx_vmem, out_vmem):
            @pl.loop(0, SC_BLOCK, step=2)
            def _(i):
                ridx = plsc.load_gather(idx_vmem, [zero, i + row_off])
                tflat = (ridx << 3) + cold
                vals = plsc.load_gather(table_v, [tflat >> 7, tflat & 127])
                oflat = (i << 3) + lanes
                plsc.store_scatter(out_vmem, [oflat >> 7, oflat & 127], vals)

        pltpu.emit_pipeline(
            body,
            grid=(N_PAD // SC_BLOCK,),
            in_specs=[pl.BlockSpec((1, SC_BLOCK), lambda i: (0, i))],
            out_specs=[pl.BlockSpec((SC_BLOCK // 16, 128),
                                    lambda i: (i, 0))],
            core_axis_name=("core", "subcore"),
            dimension_semantics=(pltpu.PARALLEL,),
        )(idx_hbm, out_hbm)

    return gather_kernel(table_packed, idx2d)


def _tc_body(zeb_ref, x_ref, w1_ref, w2_ref, b_ref, o_ref):
    acc = jnp.dot(zeb_ref[...], w1_ref[...],
                  preferred_element_type=jnp.float32)
    acc = acc + jnp.dot(x_ref[...], w2_ref[...],
                        preferred_element_type=jnp.float32)
    o_ref[...] = acc + b_ref[...]


def _tc_project(zeb, x, w1, w2, b2d):
    grid = (N // BT,)
    return pl.pallas_call(
        _tc_body,
        grid=grid,
        in_specs=[
            pl.BlockSpec((BT, EMB), lambda i: (i, 0)),
            pl.BlockSpec((BT, XDIM), lambda i: (i, 0)),
            pl.BlockSpec((EMB, HIDDEN), lambda i: (0, 0)),
            pl.BlockSpec((XDIM, HIDDEN), lambda i: (0, 0)),
            pl.BlockSpec((1, HIDDEN), lambda i: (0, 0)),
        ],
        out_specs=pl.BlockSpec((BT, HIDDEN), lambda i: (i, 0)),
        out_shape=jax.ShapeDtypeStruct((N, HIDDEN), jnp.float32),
    )(zeb, x, w1, w2, b2d)


def kernel(x, z, z_table, W, b):
    idx = jnp.pad(z.astype(jnp.int32), (0, N_PAD - N)).reshape(1, N_PAD)
    table_packed = jnp.pad(z_table.reshape(-1),
                           (0, TROWS_PAD * 128 - VOCAB * EMB)
                           ).reshape(TROWS_PAD, 128)
    w1 = W[:EMB]
    w2 = W[EMB:]
    b2d = b.reshape(1, HIDDEN)

    zebp = _sc_gather(table_packed, idx)
    zeb = zebp.reshape(N_PAD, EMB)
    return _tc_project(zeb, x, w1, w2, b2d)


# speedup vs baseline: 3.6483x; 1.6814x over previous
import dataclasses
import functools

import jax
import jax.numpy as jnp
from jax import lax
from jax.experimental import pallas as pl
from jax.experimental.pallas import tpu as pltpu
from jax.experimental.pallas import tpu_sc as plsc

N = 1_000_000
HIDDEN = 256
EMB = 8
XDIM = 11
VOCAB = 1000
TROWS = (VOCAB * EMB + 127) // 128
TROWS_PAD = 64

SC_CORES = 2
SC_SUBCORES = 16
SC_LANES = 16
SC_BLOCK = 1024
SC_CHUNK = SC_BLOCK * SC_CORES * SC_SUBCORES
N_PAD = ((N + SC_CHUNK - 1) // SC_CHUNK) * SC_CHUNK

BT = 4096


def _sc_gather(table_packed, idx2d):
    mesh = plsc.VectorSubcoreMesh(core_axis_name="core",
                                  subcore_axis_name="subcore")
    cp = pltpu.CompilerParams()
    if "needs_layout_passes" in pltpu.CompilerParams.__dataclass_fields__:
        cp = dataclasses.replace(cp, needs_layout_passes=False)

    @functools.partial(
        pl.kernel,
        out_type=jax.ShapeDtypeStruct((EMB, N_PAD), jnp.float32),
        mesh=mesh,
        scratch_types=[pltpu.VMEM((TROWS_PAD, 128), jnp.float32)],
        compiler_params=cp,
    )
    def gather_kernel(table_hbm, idx_hbm, out_hbm, table_v):
        pltpu.sync_copy(table_hbm, table_v)

        lanes = lax.iota(jnp.int32, SC_LANES)
        row_off = lanes >> 3
        cold = lanes & 7
        zero = jnp.zeros((SC_LANES,), jnp.int32)

        def body(idx_vmem, out_vmem):
            @pl.loop(0, SC_BLOCK, step=2)
            def _(i):
                ridx = plsc.load_gather(idx_vmem, [zero, i + row_off])
                tflat = (ridx << 3) + cold
                vals = plsc.load_gather(table_v, [tflat >> 7, tflat & 127])
                plsc.store_scatter(out_vmem, [cold, i + row_off], vals)

        pltpu.emit_pipeline(
            body,
            grid=(N_PAD // SC_BLOCK,),
            in_specs=[pl.BlockSpec((1, SC_BLOCK), lambda i: (0, i))],
            out_specs=[pl.BlockSpec((EMB, SC_BLOCK), lambda i: (0, i))],
            core_axis_name=("core", "subcore"),
            dimension_semantics=(pltpu.PARALLEL,),
        )(idx_hbm, out_hbm)

    return gather_kernel(table_packed, idx2d)


def _tc_body(zebt_ref, x_ref, w1_ref, w2_ref, b_ref, o_ref):
    acc = lax.dot_general(zebt_ref[...], w1_ref[...],
                          dimension_numbers=(((0,), (0,)), ((), ())),
                          preferred_element_type=jnp.float32)
    acc = acc + jnp.dot(x_ref[...], w2_ref[...],
                        preferred_element_type=jnp.float32)
    o_ref[...] = acc + b_ref[...]


def _tc_project(zebt, x, w1, w2, b2d):
    grid = ((N + BT - 1) // BT,)
    return pl.pallas_call(
        _tc_body,
        grid=grid,
        in_specs=[
            pl.BlockSpec((EMB, BT), lambda i: (0, i)),
            pl.BlockSpec((BT, XDIM), lambda i: (i, 0)),
            pl.BlockSpec((EMB, HIDDEN), lambda i: (0, 0)),
            pl.BlockSpec((XDIM, HIDDEN), lambda i: (0, 0)),
            pl.BlockSpec((1, HIDDEN), lambda i: (0, 0)),
        ],
        out_specs=pl.BlockSpec((BT, HIDDEN), lambda i: (i, 0)),
        out_shape=jax.ShapeDtypeStruct((N, HIDDEN), jnp.float32),
    )(zebt, x, w1, w2, b2d)


def kernel(x, z, z_table, W, b):
    idx = jnp.pad(z.astype(jnp.int32), (0, N_PAD - N)).reshape(1, N_PAD)
    table_packed = jnp.pad(z_table.reshape(-1),
                           (0, TROWS_PAD * 128 - VOCAB * EMB)
                           ).reshape(TROWS_PAD, 128)
    w1 = W[:EMB]
    w2 = W[EMB:]
    b2d = b.reshape(1, HIDDEN)

    zebt = _sc_gather(table_packed, idx)
    return _tc_project(zebt, x, w1, w2, b2d)
